# Initial kernel scaffold; baseline (speedup 1.0000x reference)
import jax
import jax.numpy as jnp
from jax.experimental import pallas as pl


def kernel(in_feat, edge_index, W1, b1, W2, b2):
    raise NotImplementedError("write your pallas kernel here")


# speedup vs baseline: 9.5801x; 9.5801x over previous
import functools

import jax
import jax.numpy as jnp
from jax import lax
from jax.experimental import pallas as pl
from jax.experimental.pallas import tpu as pltpu
from jax.experimental.pallas import tpu_sc as plsc

N = 10000
E = 320000
D_IN = 128
D_H = 128
D_OUT = 64

NC = 2
NS = 16
NW = NC * NS
CH = 80
NCH = E // (NW * CH)
WR = 632
WR_LAST = N - 15 * WR
DEG_W = 16

_MESH = plsc.VectorSubcoreMesh(
    core_axis_name="c", subcore_axis_name="s", num_cores=NC, num_subcores=NS
)


@functools.partial(
    pl.kernel,
    out_type=(
        jax.ShapeDtypeStruct((NC, N, DEG_W), jnp.float32),
        jax.ShapeDtypeStruct((NC, N, DEG_W), jnp.float32),
    ),
    mesh=_MESH,
    compiler_params=pltpu.CompilerParams(use_tc_tiling_on_sc=False),
    scratch_types=[
        pltpu.VMEM_SHARED((N, DEG_W), jnp.float32),
        pltpu.VMEM_SHARED((N, DEG_W), jnp.float32),
        pltpu.VMEM((CH,), jnp.int32),
        pltpu.VMEM((CH,), jnp.int32),
        pltpu.VMEM((CH,), jnp.int32),
        pltpu.VMEM((CH,), jnp.int32),
        pltpu.VMEM((CH, DEG_W), jnp.float32),
        pltpu.SemaphoreType.DMA,
        pltpu.SemaphoreType.DMA,
    ],
)
def _deg_kernel(src1d, dst1d, z16, ones, dout, din,
                acc_o, acc_i, is0, is1, id0, id1, ones_v, semi0, semi1):
    c = lax.axis_index("c")
    s = lax.axis_index("s")
    wid = c * NS + s
    ebase = wid * NCH * CH
    pltpu.sync_copy(ones, ones_v)

    isbuf = (is0, is1)
    idbuf = (id0, id1)
    semi = (semi0, semi1)

    def load_idx(j, p):
        pltpu.async_copy(src1d.at[pl.ds(ebase + j * CH, CH)], isbuf[p], semi[p])
        pltpu.async_copy(dst1d.at[pl.ds(ebase + j * CH, CH)], idbuf[p], semi[p])

    def wait_idx(j, p):
        pltpu.make_async_copy(src1d.at[pl.ds(ebase + j * CH, CH)], isbuf[p], semi[p]).wait()
        pltpu.make_async_copy(dst1d.at[pl.ds(ebase + j * CH, CH)], idbuf[p], semi[p]).wait()

    @pl.when(s == 0)
    def _():
        pltpu.sync_copy(z16, acc_o)
        pltpu.sync_copy(z16, acc_i)

    load_idx(0, 0)
    load_idx(1, 1)
    plsc.subcore_barrier()

    def body(j, _):
        for p in (0, 1):
            @pl.when(j % 2 == p)
            def _():
                wait_idx(j, p)
                pltpu.sync_copy(ones_v, acc_o.at[isbuf[p]], add=True)
                pltpu.sync_copy(ones_v, acc_i.at[idbuf[p]], add=True)

                @pl.when(j + 2 < NCH)
                def _():
                    load_idx(j + 2, p)

        return ()

    lax.fori_loop(0, NCH, body, ())
    plsc.subcore_barrier()

    def writeout(nrows):
        sl = pl.ds(s * WR, nrows)
        pltpu.sync_copy(acc_o.at[sl], dout.at[c, sl])
        pltpu.sync_copy(acc_i.at[sl], din.at[c, sl])

    @pl.when(s < NS - 1)
    def _():
        writeout(WR)

    @pl.when(s == NS - 1)
    def _():
        writeout(WR_LAST)


def _make_agg(D):
    @functools.partial(
        pl.kernel,
        out_type=jax.ShapeDtypeStruct((NC, N, D), jnp.float32),
        mesh=_MESH,
        compiler_params=pltpu.CompilerParams(use_tc_tiling_on_sc=False),
        scratch_types=[
            pltpu.VMEM_SHARED((N, D), jnp.float32),
            pltpu.VMEM((CH,), jnp.int32),
            pltpu.VMEM((CH,), jnp.int32),
            pltpu.VMEM((CH,), jnp.int32),
            pltpu.VMEM((CH,), jnp.int32),
            pltpu.VMEM((CH, D), jnp.float32),
            pltpu.VMEM((CH, D), jnp.float32),
            pltpu.SemaphoreType.DMA,
            pltpu.SemaphoreType.DMA,
            pltpu.SemaphoreType.DMA,
            pltpu.SemaphoreType.DMA,
        ],
    )
    def agg(h, src1d, dst1d, zd, out, acc,
            is0, is1, id0, id1, rows0, rows1, semi0, semi1, semg0, semg1):
        c = lax.axis_index("c")
        s = lax.axis_index("s")
        wid = c * NS + s
        ebase = wid * NCH * CH

        isbuf = (is0, is1)
        idbuf = (id0, id1)
        rows = (rows0, rows1)
        semi = (semi0, semi1)
        semg = (semg0, semg1)

        def load_idx(j, p):
            pltpu.async_copy(src1d.at[pl.ds(ebase + j * CH, CH)], isbuf[p], semi[p])
            pltpu.async_copy(dst1d.at[pl.ds(ebase + j * CH, CH)], idbuf[p], semi[p])

        def wait_idx(j, p):
            pltpu.make_async_copy(src1d.at[pl.ds(ebase + j * CH, CH)], isbuf[p], semi[p]).wait()
            pltpu.make_async_copy(dst1d.at[pl.ds(ebase + j * CH, CH)], idbuf[p], semi[p]).wait()

        def gather(p):
            pltpu.async_copy(h.at[isbuf[p]], rows[p], semg[p])

        def wait_gather(p):
            pltpu.make_async_copy(h.at[isbuf[p]], rows[p], semg[p]).wait()

        @pl.when(s == 0)
        def _():
            pltpu.sync_copy(zd, acc)

        load_idx(0, 0)
        load_idx(1, 1)
        plsc.subcore_barrier()
        wait_idx(0, 0)
        gather(0)

        def body(j, _):
            for p in (0, 1):
                @pl.when(j % 2 == p)
                def _():
                    q = 1 - p

                    @pl.when(j + 1 < NCH)
                    def _():
                        wait_idx(j + 1, q)
                        gather(q)

                    wait_gather(p)
                    pltpu.sync_copy(rows[p], acc.at[idbuf[p]], add=True)

                    @pl.when(j + 2 < NCH)
                    def _():
                        load_idx(j + 2, p)

            return ()

        lax.fori_loop(0, NCH, body, ())
        plsc.subcore_barrier()

        @pl.when(s < NS - 1)
        def _():
            sl = pl.ds(s * WR, WR)
            pltpu.sync_copy(acc.at[sl], out.at[c, sl])

        @pl.when(s == NS - 1)
        def _():
            sl = pl.ds(s * WR, WR_LAST)
            pltpu.sync_copy(acc.at[sl], out.at[c, sl])

    return agg


_agg128 = _make_agg(D_H)
_agg64 = _make_agg(D_OUT)


_BLK = 1000


def _norm_from(dp, col_sum):
    deg = dp[0, :, 0] + dp[1, :, 0]
    return lax.rsqrt(jnp.maximum(deg, 1.0))


def _mm_scale(x, w, degp_out):
    m, k = x.shape
    d = w.shape[1]

    def body(x_ref, w_ref, dp_ref, o_ref):
        xw = jnp.dot(x_ref[...], w_ref[...], preferred_element_type=jnp.float32)
        nsrc = _norm_from(dp_ref[...], 0)
        o_ref[...] = xw * nsrc[:, None]

    return pl.pallas_call(
        body,
        grid=(m // _BLK,),
        in_specs=[
            pl.BlockSpec((_BLK, k), lambda i: (i, 0)),
            pl.BlockSpec((k, d), lambda i: (0, 0)),
            pl.BlockSpec((NC, _BLK, DEG_W), lambda i: (0, i, 0)),
        ],
        out_specs=pl.BlockSpec((_BLK, d), lambda i: (i, 0)),
        out_shape=jax.ShapeDtypeStruct((m, d), jnp.float32),
    )(x, w, degp_out)


def _layer2_in(aggp, degp_in, degp_out, b1r, w2):
    d = w2.shape[1]

    def body(ap_ref, di_ref, do_ref, b_ref, w_ref, o_ref):
        ap = ap_ref[...]
        agg = ap[0] + ap[1]
        ndst = _norm_from(di_ref[...], 0)
        z = agg * ndst[:, None] + b_ref[...]
        z = jnp.maximum(z, 0.0)
        nsrc = _norm_from(do_ref[...], 0)
        z = z * nsrc[:, None]
        o_ref[...] = jnp.dot(z, w_ref[...], preferred_element_type=jnp.float32)

    return pl.pallas_call(
        body,
        grid=(N // _BLK,),
        in_specs=[
            pl.BlockSpec((NC, _BLK, D_H), lambda i: (0, i, 0)),
            pl.BlockSpec((NC, _BLK, DEG_W), lambda i: (0, i, 0)),
            pl.BlockSpec((NC, _BLK, DEG_W), lambda i: (0, i, 0)),
            pl.BlockSpec((1, D_H), lambda i: (0, 0)),
            pl.BlockSpec((D_H, d), lambda i: (0, 0)),
        ],
        out_specs=pl.BlockSpec((_BLK, d), lambda i: (i, 0)),
        out_shape=jax.ShapeDtypeStruct((N, d), jnp.float32),
    )(aggp, degp_in, degp_out, b1r, w2)


def _final(aggp, degp_in, b2r):
    def body(ap_ref, di_ref, b_ref, o_ref):
        ap = ap_ref[...]
        agg = ap[0] + ap[1]
        ndst = _norm_from(di_ref[...], 0)
        o_ref[...] = agg * ndst[:, None] + b_ref[...]

    return pl.pallas_call(
        body,
        grid=(N // _BLK,),
        in_specs=[
            pl.BlockSpec((NC, _BLK, D_OUT), lambda i: (0, i, 0)),
            pl.BlockSpec((NC, _BLK, DEG_W), lambda i: (0, i, 0)),
            pl.BlockSpec((1, D_OUT), lambda i: (0, 0)),
        ],
        out_specs=pl.BlockSpec((_BLK, D_OUT), lambda i: (i, 0)),
        out_shape=jax.ShapeDtypeStruct((N, D_OUT), jnp.float32),
    )(aggp, degp_in, b2r)


def kernel(in_feat, edge_index, W1, b1, W2, b2):
    src1d = edge_index[0]
    dst1d = edge_index[1]
    z16 = jnp.zeros((N, DEG_W), jnp.float32)
    ones = jnp.ones((CH, DEG_W), jnp.float32)
    z128 = jnp.zeros((N, D_H), jnp.float32)
    z64 = jnp.zeros((N, D_OUT), jnp.float32)

    degp_out, degp_in = _deg_kernel(src1d, dst1d, z16, ones)
    h1 = _mm_scale(in_feat, W1, degp_out)
    agg1 = _agg128(h1, src1d, dst1d, z128)
    h2 = _layer2_in(agg1, degp_in, degp_out, b1.reshape(1, D_H), W2)
    agg2 = _agg64(h2, src1d, dst1d, z64)
    return _final(agg2, degp_in, b2.reshape(1, D_OUT))
